# Initial kernel scaffold; baseline (speedup 1.0000x reference)
import functools

import jax
import jax.numpy as jnp
from jax import lax
from jax.experimental import pallas as pl
from jax.experimental.pallas import tpu as pltpu
from jax.experimental.pallas import tpu_sc as plsc

N_NODES = 10000
N_EDGES = 320000
D = 128

NC = 2
NS = 16
NW = NC * NS

B = 128
EROWS = N_EDGES // B
ROWS_PER_TILE = N_NODES // NS
CNT_PAD = 16 * 640


def _sc_body(x_hbm, col_hbm, row_hbm, acc_out, cnt_out,
             col_v, row_v, rows_v, ones_v, zcnt_v, gsem):
    c = lax.axis_index("c")
    s = lax.axis_index("s")
    w = s * NC + c

    def _zero_rows(i, carry):
        for j in range(D // 16):
            rows_v[i, pl.ds(j * 16, 16)] = jnp.zeros((16,), jnp.float32)
        return carry
    lax.fori_loop(0, B, _zero_rows, 0)
    for j in range(B // 16):
        ones_v[pl.ds(j * 16, 16)] = jnp.ones((16,), jnp.float32)
    def _zero_cnt(i, carry):
        zcnt_v[pl.ds(pl.multiple_of(i * 16, 16), 16)] = jnp.zeros((16,), jnp.float32)
        return carry
    lax.fori_loop(0, 640 // 16, _zero_cnt, 0)

    acc_sh = acc_out.at[c]
    cnt_sh = cnt_out.at[c]
    for q in range(5):
        pltpu.sync_copy(rows_v.at[pl.ds(0, 125), :],
                        acc_sh.at[pl.ds(s * ROWS_PER_TILE + q * 125, 125), :])
    pltpu.sync_copy(zcnt_v, cnt_sh.at[pl.ds(pl.multiple_of(s * 640, 8), 640)])
    plsc.subcore_barrier()

    n_w = jnp.where(w < (EROWS % NW), EROWS // NW + 1, EROWS // NW)

    def _edge_step(k, carry):
        g = w + k * NW
        pltpu.sync_copy(col_hbm.at[g], col_v)
        pltpu.sync_copy(row_hbm.at[g], row_v)
        pltpu.async_copy(x_hbm.at[row_v], rows_v, gsem).wait()
        pltpu.sync_copy(rows_v, acc_sh.at[col_v], add=True)
        pltpu.sync_copy(ones_v, cnt_sh.at[col_v], add=True)
        return carry
    lax.fori_loop(0, n_w, _edge_step, 0)

    plsc.subcore_barrier()


def _sc_partials(x, col2, row2):
    kfn = pl.kernel(
        _sc_body,
        out_type=[
            jax.ShapeDtypeStruct((NC, N_NODES, D), jnp.float32),
            jax.ShapeDtypeStruct((NC, CNT_PAD), jnp.float32),
        ],
        mesh=plsc.VectorSubcoreMesh(core_axis_name="c", subcore_axis_name="s"),
        scratch_types=[
            pltpu.VMEM((B,), jnp.int32),
            pltpu.VMEM((B,), jnp.int32),
            pltpu.VMEM((B, D), jnp.float32),
            pltpu.VMEM((B,), jnp.float32),
            pltpu.VMEM((640,), jnp.float32),
            pltpu.SemaphoreType.DMA,
        ],
    )
    return kfn(x, col2, row2)


def _combine_body(acc_ref, cnt_ref, x_ref, o_ref):
    ssum = acc_ref[0] + acc_ref[1]
    cnt = cnt_ref[0] + cnt_ref[1]
    denom = jnp.maximum(cnt, 1.0)
    o_ref[:, :D] = ssum / denom[:, None]
    o_ref[:, D:] = jnp.where((cnt > 0.0)[:, None], x_ref[...], 0.0)


def _combine(acc, cnt, x):
    R = 1000
    grid = (N_NODES // R,)
    return pl.pallas_call(
        _combine_body,
        grid=grid,
        in_specs=[
            pl.BlockSpec((NC, R, D), lambda i: (0, i, 0)),
            pl.BlockSpec((NC, R), lambda i: (0, i)),
            pl.BlockSpec((R, D), lambda i: (i, 0)),
        ],
        out_specs=pl.BlockSpec((R, 2 * D), lambda i: (i, 0)),
        out_shape=jax.ShapeDtypeStruct((N_NODES, 2 * D), jnp.float32),
    )(acc, cnt, x)


def kernel(x, es):
    es32 = es.astype(jnp.int32)
    col2 = es32[0].reshape(EROWS, B)
    row2 = es32[1].reshape(EROWS, B)
    acc, cnt_pad = _sc_partials(x, col2, row2)
    return _combine(acc, cnt_pad[:, :N_NODES], x)


# speedup vs baseline: 12.1623x; 12.1623x over previous
import functools

import jax
import jax.numpy as jnp
from jax import lax
from jax.experimental import pallas as pl
from jax.experimental.pallas import tpu as pltpu
from jax.experimental.pallas import tpu_sc as plsc

N_NODES = 10000
N_EDGES = 320000
D = 128

NC = 2
NS = 16
NW = NC * NS

B = 128
EROWS = N_EDGES // B
N_ACC = 16 * 640
ROWS_PER_TILE = N_ACC // NS
CNT_PAD = N_ACC


def _sc_body(x_hbm, col_hbm, row_hbm, acc_out, cnt_out,
             acc_sh, cnt_sh, col_v, row_v, rows_v, ones_v, zcnt_v, gsem):
    c = lax.axis_index("c")
    s = lax.axis_index("s")
    w = s * NC + c

    def _zero_rows(i, carry):
        for j in range(D // 16):
            rows_v[i, pl.ds(j * 16, 16)] = jnp.zeros((16,), jnp.float32)
        return carry
    lax.fori_loop(0, B, _zero_rows, 0)
    for j in range(B // 16):
        ones_v[pl.ds(j * 16, 16)] = jnp.ones((16,), jnp.float32)
    def _zero_cnt(i, carry):
        zcnt_v[pl.ds(pl.multiple_of(i * 16, 16), 16)] = jnp.zeros((16,), jnp.float32)
        return carry
    lax.fori_loop(0, 640 // 16, _zero_cnt, 0)

    for q in range(ROWS_PER_TILE // B):
        pltpu.sync_copy(rows_v,
                        acc_sh.at[pl.ds(pl.multiple_of(s * ROWS_PER_TILE + q * B, 8), B), :])
    pltpu.sync_copy(zcnt_v, cnt_sh.at[pl.ds(pl.multiple_of(s * 640, 8), 640)])
    plsc.subcore_barrier()

    n_w = jnp.where(w < (EROWS % NW), EROWS // NW + 1, EROWS // NW)

    def _edge_step(k, carry):
        g = w + k * NW
        pltpu.sync_copy(col_hbm.at[g], col_v)
        pltpu.sync_copy(row_hbm.at[g], row_v)
        pltpu.async_copy(x_hbm.at[row_v], rows_v, gsem).wait()
        pltpu.sync_copy(rows_v, acc_sh.at[col_v], add=True)
        pltpu.sync_copy(ones_v, cnt_sh.at[col_v], add=True)
        return carry
    lax.fori_loop(0, n_w, _edge_step, 0)

    plsc.subcore_barrier()

    pltpu.sync_copy(acc_sh.at[pl.ds(pl.multiple_of(s * ROWS_PER_TILE, 8), ROWS_PER_TILE), :],
                    acc_out.at[c, pl.ds(pl.multiple_of(s * ROWS_PER_TILE, 8), ROWS_PER_TILE), :])
    pltpu.sync_copy(cnt_sh.at[pl.ds(pl.multiple_of(s * 640, 8), 640)],
                    cnt_out.at[c, pl.ds(pl.multiple_of(s * 640, 8), 640)])


def _sc_partials(x, col2, row2):
    kfn = pl.kernel(
        _sc_body,
        out_type=[
            jax.ShapeDtypeStruct((NC, N_ACC, D), jnp.float32),
            jax.ShapeDtypeStruct((NC, CNT_PAD), jnp.float32),
        ],
        mesh=plsc.VectorSubcoreMesh(core_axis_name="c", subcore_axis_name="s"),
        scratch_types=[
            pltpu.VMEM_SHARED((N_ACC, D), jnp.float32),
            pltpu.VMEM_SHARED((CNT_PAD,), jnp.float32),
            pltpu.VMEM((B,), jnp.int32),
            pltpu.VMEM((B,), jnp.int32),
            pltpu.VMEM((B, D), jnp.float32),
            pltpu.VMEM((B,), jnp.float32),
            pltpu.VMEM((640,), jnp.float32),
            pltpu.SemaphoreType.DMA,
        ],
    )
    return kfn(x, col2, row2)


def _combine_body(acc_ref, cnt_ref, x_ref, o_ref):
    ssum = acc_ref[0] + acc_ref[1]
    cnt = cnt_ref[0] + cnt_ref[1]
    denom = jnp.maximum(cnt, 1.0)
    o_ref[:, :D] = ssum / denom
    o_ref[:, D:] = jnp.where(cnt > 0.0, x_ref[...], 0.0)


def _combine(acc, cnt, x):
    R = 1000
    grid = (N_NODES // R,)
    return pl.pallas_call(
        _combine_body,
        grid=grid,
        in_specs=[
            pl.BlockSpec((NC, R, D), lambda i: (0, i, 0)),
            pl.BlockSpec((NC, R, 1), lambda i: (0, i, 0)),
            pl.BlockSpec((R, D), lambda i: (i, 0)),
        ],
        out_specs=pl.BlockSpec((R, 2 * D), lambda i: (i, 0)),
        out_shape=jax.ShapeDtypeStruct((N_NODES, 2 * D), jnp.float32),
    )(acc, cnt, x)


def kernel(x, es):
    es32 = es.astype(jnp.int32)
    col2 = es32[0].reshape(EROWS, B)
    row2 = es32[1].reshape(EROWS, B)
    acc, cnt_pad = _sc_partials(x, col2, row2)
    cnt = cnt_pad[:, :N_NODES].reshape(NC, N_NODES, 1)
    return _combine(acc[:, :N_NODES, :], cnt, x)


# speedup vs baseline: 18.0528x; 1.4843x over previous
import functools

import jax
import jax.numpy as jnp
from jax import lax
from jax.experimental import pallas as pl
from jax.experimental.pallas import tpu as pltpu
from jax.experimental.pallas import tpu_sc as plsc

N_NODES = 10000
N_EDGES = 320000
D = 128

NC = 2
NS = 16
NW = NC * NS

B = 128
EROWS = N_EDGES // B
N_ACC = 16 * 640
ROWS_PER_TILE = N_ACC // NS
CNT_PAD = N_ACC


NB = 2
R_MAX = -(-(EROWS // NW + 1) // NB)


def _sc_body(x_hbm, col_hbm, row_hbm, acc_out, cnt_out,
             acc_sh, cnt_sh, col_v, row_v, rows_v, ones_v, zcnt_v,
             gsem, ssem, csem):
    c = lax.axis_index("c")
    s = lax.axis_index("s")
    w = s * NC + c

    def _zero_rows(i, carry):
        for j in range(D // 16):
            rows_v[0, i, pl.ds(j * 16, 16)] = jnp.zeros((16,), jnp.float32)
        return carry
    lax.fori_loop(0, B, _zero_rows, 0)
    for j in range(B // 16):
        ones_v[pl.ds(j * 16, 16)] = jnp.ones((16,), jnp.float32)
    def _zero_cnt(i, carry):
        zcnt_v[pl.ds(pl.multiple_of(i * 16, 16), 16)] = jnp.zeros((16,), jnp.float32)
        return carry
    lax.fori_loop(0, 640 // 16, _zero_cnt, 0)

    for q in range(ROWS_PER_TILE // B):
        pltpu.sync_copy(rows_v.at[0],
                        acc_sh.at[pl.ds(pl.multiple_of(s * ROWS_PER_TILE + q * B, 8), B), :])
    pltpu.sync_copy(zcnt_v, cnt_sh.at[pl.ds(pl.multiple_of(s * 640, 8), 640)])
    plsc.subcore_barrier()

    n_w = jnp.where(w < (EROWS % NW), EROWS // NW + 1, EROWS // NW)

    def _load_and_gather(k, b):
        g = w + k * NW
        pltpu.sync_copy(col_hbm.at[g], col_v.at[b])
        pltpu.sync_copy(row_hbm.at[g], row_v.at[b])
        pltpu.async_copy(x_hbm.at[row_v.at[b]], rows_v.at[b], gsem.at[b])

    for b in range(NB):
        _load_and_gather(jnp.int32(b), b)

    def _round(r, carry):
        for b in range(NB):
            k = r * NB + b
            @pl.when(k < n_w)
            def _():
                pltpu.make_async_copy(x_hbm.at[row_v.at[b]], rows_v.at[b],
                                      gsem.at[b]).wait()
                pltpu.async_copy(rows_v.at[b], acc_sh.at[col_v.at[b]],
                                 ssem.at[b], add=True)
                pltpu.async_copy(ones_v, cnt_sh.at[col_v.at[b]],
                                 csem.at[b], add=True)
        for b in range(NB):
            k = r * NB + b
            @pl.when(k < n_w)
            def _():
                pltpu.make_async_copy(rows_v.at[b], acc_sh.at[col_v.at[b]],
                                      ssem.at[b]).wait()
                pltpu.make_async_copy(ones_v, cnt_sh.at[col_v.at[b]],
                                      csem.at[b]).wait()
            @pl.when(k + NB < n_w)
            def _():
                _load_and_gather(k + NB, b)
        return carry
    lax.fori_loop(0, R_MAX, _round, 0)

    plsc.subcore_barrier()

    pltpu.sync_copy(acc_sh.at[pl.ds(pl.multiple_of(s * ROWS_PER_TILE, 8), ROWS_PER_TILE), :],
                    acc_out.at[c, pl.ds(pl.multiple_of(s * ROWS_PER_TILE, 8), ROWS_PER_TILE), :])
    pltpu.sync_copy(cnt_sh.at[pl.ds(pl.multiple_of(s * 640, 8), 640)],
                    cnt_out.at[c, pl.ds(pl.multiple_of(s * 640, 8), 640)])


def _sc_partials(x, col2, row2):
    kfn = pl.kernel(
        _sc_body,
        out_type=[
            jax.ShapeDtypeStruct((NC, N_ACC, D), jnp.float32),
            jax.ShapeDtypeStruct((NC, CNT_PAD), jnp.float32),
        ],
        mesh=plsc.VectorSubcoreMesh(core_axis_name="c", subcore_axis_name="s"),
        scratch_types=[
            pltpu.VMEM_SHARED((N_ACC, D), jnp.float32),
            pltpu.VMEM_SHARED((CNT_PAD,), jnp.float32),
            pltpu.VMEM((NB, B), jnp.int32),
            pltpu.VMEM((NB, B), jnp.int32),
            pltpu.VMEM((NB, B, D), jnp.float32),
            pltpu.VMEM((B,), jnp.float32),
            pltpu.VMEM((640,), jnp.float32),
            pltpu.SemaphoreType.DMA((NB,)),
            pltpu.SemaphoreType.DMA((NB,)),
            pltpu.SemaphoreType.DMA((NB,)),
        ],
    )
    return kfn(x, col2, row2)


def _combine_body(acc_ref, cnt_ref, x_ref, o_ref):
    ssum = acc_ref[0] + acc_ref[1]
    cnt = cnt_ref[0] + cnt_ref[1]
    denom = jnp.maximum(cnt, 1.0)
    o_ref[:, :D] = ssum / denom
    o_ref[:, D:] = jnp.where(cnt > 0.0, x_ref[...], 0.0)


def _combine(acc, cnt, x):
    R = 1000
    grid = (N_NODES // R,)
    return pl.pallas_call(
        _combine_body,
        grid=grid,
        in_specs=[
            pl.BlockSpec((NC, R, D), lambda i: (0, i, 0)),
            pl.BlockSpec((NC, R, 1), lambda i: (0, i, 0)),
            pl.BlockSpec((R, D), lambda i: (i, 0)),
        ],
        out_specs=pl.BlockSpec((R, 2 * D), lambda i: (i, 0)),
        out_shape=jax.ShapeDtypeStruct((N_NODES, 2 * D), jnp.float32),
    )(acc, cnt, x)


def kernel(x, es):
    es32 = es.astype(jnp.int32)
    col2 = es32[0].reshape(EROWS, B)
    row2 = es32[1].reshape(EROWS, B)
    acc, cnt_pad = _sc_partials(x, col2, row2)
    cnt = cnt_pad[:, :N_NODES].reshape(NC, N_NODES, 1)
    return _combine(acc[:, :N_NODES, :], cnt, x)


# speedup vs baseline: 18.5632x; 1.0283x over previous
import functools

import jax
import jax.numpy as jnp
from jax import lax
from jax.experimental import pallas as pl
from jax.experimental.pallas import tpu as pltpu
from jax.experimental.pallas import tpu_sc as plsc

N_NODES = 10000
N_EDGES = 320000
D = 128

NC = 2
NS = 16
NW = NC * NS

B = 128
EROWS = N_EDGES // B
N_ACC = 16 * 640
ROWS_PER_TILE = N_ACC // NS
CNT_PAD = N_ACC


NB = 2
R_MAX = -(-(EROWS // NW + 1) // NB)


def _sc_body(x_hbm, col_hbm, row_hbm, acc_out, cnt_out,
             acc_sh, cnt_sh, col_v, row_v, rows_v, ones_v, zcnt_v,
             gsem, ssem, csem):
    c = lax.axis_index("c")
    s = lax.axis_index("s")
    w = s * NC + c

    def _zero_rows(i, carry):
        for j in range(D // 16):
            rows_v[0, i, pl.ds(j * 16, 16)] = jnp.zeros((16,), jnp.float32)
        return carry
    lax.fori_loop(0, B, _zero_rows, 0)
    for j in range(B // 16):
        ones_v[pl.ds(j * 16, 16)] = jnp.ones((16,), jnp.float32)
    def _zero_cnt(i, carry):
        zcnt_v[pl.ds(pl.multiple_of(i * 16, 16), 16)] = jnp.zeros((16,), jnp.float32)
        return carry
    lax.fori_loop(0, 640 // 16, _zero_cnt, 0)

    for q in range(ROWS_PER_TILE // B):
        pltpu.sync_copy(rows_v.at[0],
                        acc_sh.at[pl.ds(pl.multiple_of(s * ROWS_PER_TILE + q * B, 8), B), :])
    pltpu.sync_copy(zcnt_v, cnt_sh.at[pl.ds(pl.multiple_of(s * 640, 8), 640)])
    plsc.subcore_barrier()

    n_w = jnp.where(w < (EROWS % NW), EROWS // NW + 1, EROWS // NW)

    def _load_and_gather(k, b):
        g = w + k * NW
        pltpu.sync_copy(col_hbm.at[g], col_v.at[b])
        pltpu.sync_copy(row_hbm.at[g], row_v.at[b])
        pltpu.async_copy(x_hbm.at[row_v.at[b]], rows_v.at[b], gsem.at[b])

    for b in range(NB):
        _load_and_gather(jnp.int32(b), b)

    def _round(r, carry):
        for b in range(NB):
            k = r * NB + b
            @pl.when(k < n_w)
            def _():
                pltpu.make_async_copy(x_hbm.at[row_v.at[b]], rows_v.at[b],
                                      gsem.at[b]).wait()
                pltpu.async_copy(rows_v.at[b], acc_sh.at[col_v.at[b]],
                                 ssem.at[b], add=True)
                pltpu.async_copy(ones_v, cnt_sh.at[col_v.at[b]],
                                 csem.at[b], add=True)
        for b in range(NB):
            k = r * NB + b
            @pl.when(k < n_w)
            def _():
                pltpu.make_async_copy(rows_v.at[b], acc_sh.at[col_v.at[b]],
                                      ssem.at[b]).wait()
                pltpu.make_async_copy(ones_v, cnt_sh.at[col_v.at[b]],
                                      csem.at[b]).wait()
            @pl.when(k + NB < n_w)
            def _():
                _load_and_gather(k + NB, b)
        return carry
    lax.fori_loop(0, R_MAX, _round, 0)

    plsc.subcore_barrier()

    pltpu.sync_copy(acc_sh.at[pl.ds(pl.multiple_of(s * ROWS_PER_TILE, 8), ROWS_PER_TILE), :],
                    acc_out.at[c, pl.ds(pl.multiple_of(s * ROWS_PER_TILE, 8), ROWS_PER_TILE), :])
    pltpu.sync_copy(cnt_sh.at[pl.ds(pl.multiple_of(s * 640, 8), 640)],
                    cnt_out.at[c, pl.ds(pl.multiple_of(s * 640, 8), 640)])


def _sc_partials(x, col2, row2):
    kfn = pl.kernel(
        _sc_body,
        out_type=[
            jax.ShapeDtypeStruct((NC, N_ACC, D), jnp.float32),
            jax.ShapeDtypeStruct((NC, CNT_PAD), jnp.float32),
        ],
        mesh=plsc.VectorSubcoreMesh(core_axis_name="c", subcore_axis_name="s"),
        scratch_types=[
            pltpu.VMEM_SHARED((N_ACC, D), jnp.float32),
            pltpu.VMEM_SHARED((CNT_PAD,), jnp.float32),
            pltpu.VMEM((NB, B), jnp.int32),
            pltpu.VMEM((NB, B), jnp.int32),
            pltpu.VMEM((NB, B, D), jnp.float32),
            pltpu.VMEM((B,), jnp.float32),
            pltpu.VMEM((640,), jnp.float32),
            pltpu.SemaphoreType.DMA((NB,)),
            pltpu.SemaphoreType.DMA((NB,)),
            pltpu.SemaphoreType.DMA((NB,)),
        ],
    )
    return kfn(x, col2, row2)


def _combine_body(acc_ref, cnt_ref, x_ref, o_ref):
    ssum = acc_ref[0] + acc_ref[1]
    cnt = cnt_ref[0] + cnt_ref[1]
    denom = jnp.maximum(cnt, 1.0)
    o_ref[:, :D] = ssum / denom
    o_ref[:, D:] = jnp.where(cnt > 0.0, x_ref[...], 0.0)


def _combine(acc, cnt, x):
    R = 1000
    grid = (N_NODES // R,)
    return pl.pallas_call(
        _combine_body,
        grid=grid,
        in_specs=[
            pl.BlockSpec((NC, R, D), lambda i: (0, i, 0)),
            pl.BlockSpec((NC, R, 1), lambda i: (0, i, 0)),
            pl.BlockSpec((R, D), lambda i: (i, 0)),
        ],
        out_specs=pl.BlockSpec((R, 2 * D), lambda i: (i, 0)),
        out_shape=jax.ShapeDtypeStruct((N_NODES, 2 * D), jnp.float32),
    )(acc, cnt, x)


def kernel(x, es):
    es32 = es.astype(jnp.int32)
    col2 = es32[0].reshape(EROWS, B)
    row2 = es32[1].reshape(EROWS, B)
    acc, cnt_pad = _sc_partials(x, col2, row2)
    cnt = cnt_pad[:, :N_NODES].reshape(NC, N_NODES, 1)
    return _combine(acc, cnt, x)


# speedup vs baseline: 20.8496x; 1.1232x over previous
import jax
import jax.numpy as jnp
from jax import lax
from jax.experimental import pallas as pl
from jax.experimental.pallas import tpu as pltpu
from jax.experimental.pallas import tpu_sc as plsc

N_NODES = 10000
N_EDGES = 320000
D = 128

NC = 2
NS = 16
NW = NC * NS

B = 128
EROWS = N_EDGES // B
ROWS_W = 80
PROWS = NW * ROWS_W
N_ACC = 16 * 640
ROWS_PER_TILE = N_ACC // NS
CNT_PAD = N_ACC

NB = 2
IB = 8
NSL = 3
NSLABS = ROWS_W // IB


def _sc_body(x_hbm, col_hbm, row_hbm, acc_out, cnt_out,
             acc_sh, cnt_sh, colb, rowb, rows_v, ones_v, zcnt_v,
             gsem, ssem, csem, isem):
    c = lax.axis_index("c")
    s = lax.axis_index("s")
    w = s * NC + c
    base = w * ROWS_W

    def _zero_rows(i, carry):
        for j in range(D // 16):
            rows_v[0, i, pl.ds(j * 16, 16)] = jnp.zeros((16,), jnp.float32)
        return carry
    lax.fori_loop(0, B, _zero_rows, 0)
    for j in range(B // 16):
        ones_v[pl.ds(j * 16, 16)] = jnp.ones((16,), jnp.float32)
    def _zero_cnt(i, carry):
        zcnt_v[pl.ds(pl.multiple_of(i * 16, 16), 16)] = jnp.zeros((16,), jnp.float32)
        return carry
    lax.fori_loop(0, 640 // 16, _zero_cnt, 0)

    for q in range(ROWS_PER_TILE // B):
        pltpu.sync_copy(rows_v.at[0],
                        acc_sh.at[pl.ds(pl.multiple_of(s * ROWS_PER_TILE + q * B, 8), B), :])
    pltpu.sync_copy(zcnt_v, cnt_sh.at[pl.ds(pl.multiple_of(s * 640, 8), 640)])
    plsc.subcore_barrier()

    def _slab_refs(j):
        slot = lax.rem(j, NSL)
        st = pl.multiple_of(base + j * IB, 8)
        return ((col_hbm.at[pl.ds(st, IB), :], colb.at[slot]),
                (row_hbm.at[pl.ds(st, IB), :], rowb.at[slot]))

    def _slab_load(j):
        (csrc, cdst), (rsrc, rdst) = _slab_refs(j)
        slot = lax.rem(j, NSL)
        pltpu.async_copy(csrc, cdst, isem.at[slot])
        pltpu.async_copy(rsrc, rdst, isem.at[slot])

    def _slab_wait(j):
        (csrc, cdst), (rsrc, rdst) = _slab_refs(j)
        slot = lax.rem(j, NSL)
        pltpu.make_async_copy(csrc, cdst, isem.at[slot]).wait()
        pltpu.make_async_copy(rsrc, rdst, isem.at[slot]).wait()

    def _fire_gather(slot, m, b):
        pltpu.async_copy(x_hbm.at[rowb.at[slot, m]], rows_v.at[b], gsem.at[b])

    def _wait_gather(slot, m, b):
        pltpu.make_async_copy(x_hbm.at[rowb.at[slot, m]], rows_v.at[b],
                              gsem.at[b]).wait()

    for j in range(NSL):
        _slab_load(j)
    _slab_wait(0)
    for b in range(NB):
        _fire_gather(0, b, b)

    def _slab_body(j, carry):
        jm = lax.rem(j, NSL)
        jp1 = lax.rem(j + 1, NSL)
        for m in range(0, IB, NB):
            for b in range(NB):
                _wait_gather(jm, m + b, b)
                pltpu.async_copy(rows_v.at[b], acc_sh.at[colb.at[jm, m + b]],
                                 ssem.at[b], add=True)
                pltpu.async_copy(ones_v, cnt_sh.at[colb.at[jm, m + b]],
                                 csem.at[b], add=True)
            if m == 2:
                @pl.when(j < NSLABS - 2)
                def _():
                    _slab_load(j + 2)
            if m == 6:
                @pl.when(j < NSLABS - 1)
                def _():
                    _slab_wait(j + 1)
            for b in range(NB):
                pltpu.make_async_copy(rows_v.at[b], acc_sh.at[colb.at[jm, m + b]],
                                      ssem.at[b]).wait()
                pltpu.make_async_copy(ones_v, cnt_sh.at[colb.at[jm, m + b]],
                                      csem.at[b]).wait()
                mn = m + b + NB
                if mn < IB:
                    if m == 6:
                        raise AssertionError
                    _fire_gather(jm, mn, b)
                else:
                    @pl.when(j < NSLABS - 1)
                    def _():
                        _fire_gather(jp1, mn - IB, b)
        return carry
    lax.fori_loop(0, NSLABS, _slab_body, 0)

    plsc.subcore_barrier()

    pltpu.sync_copy(acc_sh.at[pl.ds(pl.multiple_of(s * ROWS_PER_TILE, 8), ROWS_PER_TILE), :],
                    acc_out.at[c, pl.ds(pl.multiple_of(s * ROWS_PER_TILE, 8), ROWS_PER_TILE), :])
    pltpu.sync_copy(cnt_sh.at[pl.ds(pl.multiple_of(s * 640, 8), 640)],
                    cnt_out.at[c, pl.ds(pl.multiple_of(s * 640, 8), 640)])


def _sc_partials(x, col2, row2):
    kfn = pl.kernel(
        _sc_body,
        out_type=[
            jax.ShapeDtypeStruct((NC, N_ACC, D), jnp.float32),
            jax.ShapeDtypeStruct((NC, CNT_PAD), jnp.float32),
        ],
        mesh=plsc.VectorSubcoreMesh(core_axis_name="c", subcore_axis_name="s"),
        scratch_types=[
            pltpu.VMEM_SHARED((N_ACC, D), jnp.float32),
            pltpu.VMEM_SHARED((CNT_PAD,), jnp.float32),
            pltpu.VMEM((NSL, IB, B), jnp.int32),
            pltpu.VMEM((NSL, IB, B), jnp.int32),
            pltpu.VMEM((NB, B, D), jnp.float32),
            pltpu.VMEM((B,), jnp.float32),
            pltpu.VMEM((640,), jnp.float32),
            pltpu.SemaphoreType.DMA((NB,)),
            pltpu.SemaphoreType.DMA((NB,)),
            pltpu.SemaphoreType.DMA((NB,)),
            pltpu.SemaphoreType.DMA((NSL,)),
        ],
    )
    return kfn(x, col2, row2)


def _combine_body(acc_ref, cnt_ref, x_ref, o_ref):
    ssum = acc_ref[0] + acc_ref[1]
    cnt = cnt_ref[0] + cnt_ref[1]
    denom = jnp.maximum(cnt, 1.0)
    o_ref[:, :D] = ssum / denom
    o_ref[:, D:] = jnp.where(cnt > 0.0, x_ref[...], 0.0)


def _combine(acc, cnt, x):
    R = 1000
    grid = (N_NODES // R,)
    return pl.pallas_call(
        _combine_body,
        grid=grid,
        in_specs=[
            pl.BlockSpec((NC, R, D), lambda i: (0, i, 0)),
            pl.BlockSpec((NC, R, 1), lambda i: (0, i, 0)),
            pl.BlockSpec((R, D), lambda i: (i, 0)),
        ],
        out_specs=pl.BlockSpec((R, 2 * D), lambda i: (i, 0)),
        out_shape=jax.ShapeDtypeStruct((N_NODES, 2 * D), jnp.float32),
    )(acc, cnt, x)


def kernel(x, es):
    es32 = es.astype(jnp.int32)
    npad = PROWS * B - N_EDGES
    pad_i = jnp.arange(npad, dtype=jnp.int32)
    col_pad = N_NODES + (pad_i % (N_ACC - N_NODES))
    row_pad = pad_i % N_NODES
    col2 = jnp.concatenate([es32[0], col_pad]).reshape(PROWS, B)
    row2 = jnp.concatenate([es32[1], row_pad]).reshape(PROWS, B)
    acc, cnt_pad = _sc_partials(x, col2, row2)
    cnt = cnt_pad[:, :N_NODES].reshape(NC, N_NODES, 1)
    return _combine(acc, cnt, x)


# speedup vs baseline: 21.0101x; 1.0077x over previous
import jax
import jax.numpy as jnp
from jax import lax
from jax.experimental import pallas as pl
from jax.experimental.pallas import tpu as pltpu
from jax.experimental.pallas import tpu_sc as plsc

N_NODES = 10000
N_EDGES = 320000
D = 128

NC = 2
NS = 16
NW = NC * NS

B = 128
EROWS = N_EDGES // B
ROWS_W = 80
PROWS = NW * ROWS_W
N_ACC = 16 * 640
ROWS_PER_TILE = N_ACC // NS
CNT_PAD = N_ACC

NB = 2
IB = 8
NSL = 3
NSLABS = ROWS_W // IB


def _sc_body(x_hbm, col_hbm, row_hbm, acc_out, cnt_out,
             acc_sh, cnt_sh, colb, rowb, rows_v, ones_v, zcnt_v,
             gsem, ssem, csem, isem, wsem):
    c = lax.axis_index("c")
    s = lax.axis_index("s")
    w = s * NC + c
    base = w * ROWS_W

    def _slab_refs(j):
        slot = lax.rem(j, NSL)
        st = pl.multiple_of(base + j * IB, 8)
        return ((col_hbm.at[pl.ds(st, IB), :], colb.at[slot]),
                (row_hbm.at[pl.ds(st, IB), :], rowb.at[slot]))

    def _slab_load(j):
        (csrc, cdst), (rsrc, rdst) = _slab_refs(j)
        slot = lax.rem(j, NSL)
        pltpu.async_copy(csrc, cdst, isem.at[slot])
        pltpu.async_copy(rsrc, rdst, isem.at[slot])

    def _slab_wait(j):
        (csrc, cdst), (rsrc, rdst) = _slab_refs(j)
        slot = lax.rem(j, NSL)
        pltpu.make_async_copy(csrc, cdst, isem.at[slot]).wait()
        pltpu.make_async_copy(rsrc, rdst, isem.at[slot]).wait()

    for j in range(NSL):
        _slab_load(j)

    def _zero_rows(i, carry):
        for j in range(D // 16):
            rows_v[0, i, pl.ds(j * 16, 16)] = jnp.zeros((16,), jnp.float32)
        return carry
    lax.fori_loop(0, B, _zero_rows, 0)
    for j in range(B // 16):
        ones_v[pl.ds(j * 16, 16)] = jnp.ones((16,), jnp.float32)
    def _zero_cnt(i, carry):
        zcnt_v[pl.ds(pl.multiple_of(i * 16, 16), 16)] = jnp.zeros((16,), jnp.float32)
        return carry
    lax.fori_loop(0, 640 // 16, _zero_cnt, 0)

    for q in range(ROWS_PER_TILE // B):
        pltpu.sync_copy(rows_v.at[0],
                        acc_sh.at[pl.ds(pl.multiple_of(s * ROWS_PER_TILE + q * B, 8), B), :])
    pltpu.sync_copy(zcnt_v, cnt_sh.at[pl.ds(pl.multiple_of(s * 640, 8), 640)])
    plsc.subcore_barrier()

    def _fire_gather(slot, m, b):
        pltpu.async_copy(x_hbm.at[rowb.at[slot, m]], rows_v.at[b], gsem.at[b])

    def _wait_gather(slot, m, b):
        pltpu.make_async_copy(x_hbm.at[rowb.at[slot, m]], rows_v.at[b],
                              gsem.at[b]).wait()

    _slab_wait(0)
    for b in range(NB):
        _fire_gather(0, b, b)

    def _slab_body(j, carry):
        jm = lax.rem(j, NSL)
        jp1 = lax.rem(j + 1, NSL)
        for m in range(0, IB, NB):
            for b in range(NB):
                _wait_gather(jm, m + b, b)
                pltpu.async_copy(rows_v.at[b], acc_sh.at[colb.at[jm, m + b]],
                                 ssem.at[b], add=True)
                pltpu.async_copy(ones_v, cnt_sh.at[colb.at[jm, m + b]],
                                 csem.at[b], add=True)
            if m == 2:
                @pl.when(j < NSLABS - 2)
                def _():
                    _slab_load(j + 2)
            if m == 6:
                @pl.when(j < NSLABS - 1)
                def _():
                    _slab_wait(j + 1)
            for b in range(NB):
                pltpu.make_async_copy(rows_v.at[b], acc_sh.at[colb.at[jm, m + b]],
                                      ssem.at[b]).wait()
                pltpu.make_async_copy(ones_v, cnt_sh.at[colb.at[jm, m + b]],
                                      csem.at[b]).wait()
                mn = m + b + NB
                if mn < IB:
                    if m == 6:
                        raise AssertionError
                    _fire_gather(jm, mn, b)
                else:
                    @pl.when(j < NSLABS - 1)
                    def _():
                        _fire_gather(jp1, mn - IB, b)
        return carry
    lax.fori_loop(0, NSLABS, _slab_body, 0)

    plsc.subcore_barrier()

    asrc = acc_sh.at[pl.ds(pl.multiple_of(s * ROWS_PER_TILE, 8), ROWS_PER_TILE), :]
    adst = acc_out.at[c, pl.ds(pl.multiple_of(s * ROWS_PER_TILE, 8), ROWS_PER_TILE), :]
    csrc2 = cnt_sh.at[pl.ds(pl.multiple_of(s * 640, 8), 640)]
    cdst2 = cnt_out.at[c, pl.ds(pl.multiple_of(s * 640, 8), 640)]
    pltpu.async_copy(asrc, adst, wsem)
    pltpu.async_copy(csrc2, cdst2, wsem)
    pltpu.make_async_copy(asrc, adst, wsem).wait()
    pltpu.make_async_copy(csrc2, cdst2, wsem).wait()


def _sc_partials(x, col2, row2):
    kfn = pl.kernel(
        _sc_body,
        out_type=[
            jax.ShapeDtypeStruct((NC, N_ACC, D), jnp.float32),
            jax.ShapeDtypeStruct((NC, CNT_PAD), jnp.float32),
        ],
        mesh=plsc.VectorSubcoreMesh(core_axis_name="c", subcore_axis_name="s"),
        scratch_types=[
            pltpu.VMEM_SHARED((N_ACC, D), jnp.float32),
            pltpu.VMEM_SHARED((CNT_PAD,), jnp.float32),
            pltpu.VMEM((NSL, IB, B), jnp.int32),
            pltpu.VMEM((NSL, IB, B), jnp.int32),
            pltpu.VMEM((NB, B, D), jnp.float32),
            pltpu.VMEM((B,), jnp.float32),
            pltpu.VMEM((640,), jnp.float32),
            pltpu.SemaphoreType.DMA((NB,)),
            pltpu.SemaphoreType.DMA((NB,)),
            pltpu.SemaphoreType.DMA((NB,)),
            pltpu.SemaphoreType.DMA((NSL,)),
            pltpu.SemaphoreType.DMA,
        ],
    )
    return kfn(x, col2, row2)


def _combine_body(acc_ref, cnt_ref, x_ref, o_ref):
    ssum = acc_ref[0] + acc_ref[1]
    cnt = cnt_ref[0] + cnt_ref[1]
    denom = jnp.maximum(cnt, 1.0)
    o_ref[:, :D] = ssum / denom
    o_ref[:, D:] = jnp.where(cnt > 0.0, x_ref[...], 0.0)


def _combine(acc, cnt, x):
    R = 2000
    grid = (N_NODES // R,)
    return pl.pallas_call(
        _combine_body,
        grid=grid,
        in_specs=[
            pl.BlockSpec((NC, R, D), lambda i: (0, i, 0)),
            pl.BlockSpec((NC, R, 1), lambda i: (0, i, 0)),
            pl.BlockSpec((R, D), lambda i: (i, 0)),
        ],
        out_specs=pl.BlockSpec((R, 2 * D), lambda i: (i, 0)),
        out_shape=jax.ShapeDtypeStruct((N_NODES, 2 * D), jnp.float32),
    )(acc, cnt, x)


def kernel(x, es):
    es32 = es.astype(jnp.int32)
    npad = PROWS * B - N_EDGES
    pad_i = jnp.arange(npad, dtype=jnp.int32)
    col_pad = N_NODES + (pad_i % (N_ACC - N_NODES))
    row_pad = pad_i % N_NODES
    col2 = jnp.concatenate([es32[0], col_pad]).reshape(PROWS, B)
    row2 = jnp.concatenate([es32[1], row_pad]).reshape(PROWS, B)
    acc, cnt_pad = _sc_partials(x, col2, row2)
    cnt = cnt_pad[:, :N_NODES].reshape(NC, N_NODES, 1)
    return _combine(acc, cnt, x)


# speedup vs baseline: 21.0698x; 1.0028x over previous
import jax
import jax.numpy as jnp
from jax import lax
from jax.experimental import pallas as pl
from jax.experimental.pallas import tpu as pltpu
from jax.experimental.pallas import tpu_sc as plsc

N_NODES = 10000
N_EDGES = 320000
D = 128

NC = 2
NS = 16
NW = NC * NS

B = 128
EROWS = N_EDGES // B
ROWS_W = 80
PROWS = NW * ROWS_W
N_ACC = 16 * 640
ROWS_PER_TILE = N_ACC // NS
CNT_PAD = N_ACC

NB = 2
IB = 8
NSL = 3
NSLABS = ROWS_W // IB


def _sc_body(x_hbm, col_hbm, row_hbm, acc_out, cnt_out,
             acc_sh, cnt_sh, colb, rowb, rows_v, ones_v, zcnt_v,
             gsem, ssem, csem, isem, wsem):
    c = lax.axis_index("c")
    s = lax.axis_index("s")
    w = s * NC + c
    base = w * ROWS_W

    def _slab_refs(j):
        slot = lax.rem(j, NSL)
        st = pl.multiple_of(base + j * IB, 8)
        return ((col_hbm.at[pl.ds(st, IB), :], colb.at[slot]),
                (row_hbm.at[pl.ds(st, IB), :], rowb.at[slot]))

    def _slab_load(j):
        (csrc, cdst), (rsrc, rdst) = _slab_refs(j)
        slot = lax.rem(j, NSL)
        pltpu.async_copy(csrc, cdst, isem.at[slot])
        pltpu.async_copy(rsrc, rdst, isem.at[slot])

    def _slab_wait(j):
        (csrc, cdst), (rsrc, rdst) = _slab_refs(j)
        slot = lax.rem(j, NSL)
        pltpu.make_async_copy(csrc, cdst, isem.at[slot]).wait()
        pltpu.make_async_copy(rsrc, rdst, isem.at[slot]).wait()

    for j in range(NSL):
        _slab_load(j)

    def _zero_rows(i, carry):
        for j in range(D // 16):
            rows_v[0, i, pl.ds(j * 16, 16)] = jnp.zeros((16,), jnp.float32)
        return carry
    lax.fori_loop(0, B, _zero_rows, 0)
    for j in range(B // 16):
        ones_v[pl.ds(j * 16, 16)] = jnp.ones((16,), jnp.float32)
    def _zero_cnt(i, carry):
        zcnt_v[pl.ds(pl.multiple_of(i * 16, 16), 16)] = jnp.zeros((16,), jnp.float32)
        return carry
    lax.fori_loop(0, 640 // 16, _zero_cnt, 0)

    for q in range(ROWS_PER_TILE // B):
        pltpu.sync_copy(rows_v.at[0],
                        acc_sh.at[pl.ds(pl.multiple_of(s * ROWS_PER_TILE + q * B, 8), B), :])
    pltpu.sync_copy(zcnt_v, cnt_sh.at[pl.ds(pl.multiple_of(s * 640, 8), 640)])
    plsc.subcore_barrier()

    def _fire_gather(slot, m, b):
        pltpu.async_copy(x_hbm.at[rowb.at[slot, m]], rows_v.at[b], gsem.at[b])

    def _wait_gather(slot, m, b):
        pltpu.make_async_copy(x_hbm.at[rowb.at[slot, m]], rows_v.at[b],
                              gsem.at[b]).wait()

    _slab_wait(0)
    for b in range(NB):
        _fire_gather(0, b, b)

    def _slab_body(j, carry):
        jm = lax.rem(j, NSL)
        jp1 = lax.rem(j + 1, NSL)
        for m in range(0, IB, NB):
            for b in range(NB):
                _wait_gather(jm, m + b, b)
                pltpu.async_copy(rows_v.at[b], acc_sh.at[colb.at[jm, m + b]],
                                 ssem.at[b], add=True)
                pltpu.async_copy(ones_v, cnt_sh.at[colb.at[jm, m + b]],
                                 csem, add=True)
            if m == 2:
                @pl.when(j >= 1)
                def _():
                    for _i in range(IB):
                        pltpu.make_async_copy(ones_v, cnt_sh.at[colb.at[jm, 0]],
                                              csem).wait()
                @pl.when(j < NSLABS - 2)
                def _():
                    _slab_load(j + 2)
            if m == 6:
                @pl.when(j < NSLABS - 1)
                def _():
                    _slab_wait(j + 1)
            for b in range(NB):
                pltpu.make_async_copy(rows_v.at[b], acc_sh.at[colb.at[jm, m + b]],
                                      ssem.at[b]).wait()
                mn = m + b + NB
                if mn < IB:
                    if m == 6:
                        raise AssertionError
                    _fire_gather(jm, mn, b)
                else:
                    @pl.when(j < NSLABS - 1)
                    def _():
                        _fire_gather(jp1, mn - IB, b)
        return carry
    lax.fori_loop(0, NSLABS, _slab_body, 0)

    for _i in range(IB):
        pltpu.make_async_copy(ones_v, cnt_sh.at[colb.at[0, 0]], csem).wait()

    plsc.subcore_barrier()

    asrc = acc_sh.at[pl.ds(pl.multiple_of(s * ROWS_PER_TILE, 8), ROWS_PER_TILE), :]
    adst = acc_out.at[c, pl.ds(pl.multiple_of(s * ROWS_PER_TILE, 8), ROWS_PER_TILE), :]
    csrc2 = cnt_sh.at[pl.ds(pl.multiple_of(s * 640, 8), 640)]
    cdst2 = cnt_out.at[c, pl.ds(pl.multiple_of(s * 640, 8), 640)]
    pltpu.async_copy(asrc, adst, wsem)
    pltpu.async_copy(csrc2, cdst2, wsem)
    pltpu.make_async_copy(asrc, adst, wsem).wait()
    pltpu.make_async_copy(csrc2, cdst2, wsem).wait()


def _sc_partials(x, col2, row2):
    kfn = pl.kernel(
        _sc_body,
        out_type=[
            jax.ShapeDtypeStruct((NC, N_ACC, D), jnp.float32),
            jax.ShapeDtypeStruct((NC, CNT_PAD), jnp.float32),
        ],
        mesh=plsc.VectorSubcoreMesh(core_axis_name="c", subcore_axis_name="s"),
        scratch_types=[
            pltpu.VMEM_SHARED((N_ACC, D), jnp.float32),
            pltpu.VMEM_SHARED((CNT_PAD,), jnp.float32),
            pltpu.VMEM((NSL, IB, B), jnp.int32),
            pltpu.VMEM((NSL, IB, B), jnp.int32),
            pltpu.VMEM((NB, B, D), jnp.float32),
            pltpu.VMEM((B,), jnp.float32),
            pltpu.VMEM((640,), jnp.float32),
            pltpu.SemaphoreType.DMA((NB,)),
            pltpu.SemaphoreType.DMA((NB,)),
            pltpu.SemaphoreType.DMA,
            pltpu.SemaphoreType.DMA((NSL,)),
            pltpu.SemaphoreType.DMA,
        ],
    )
    return kfn(x, col2, row2)


def _combine_body(acc_ref, cnt_ref, x_ref, o_ref):
    ssum = acc_ref[0] + acc_ref[1]
    cnt = cnt_ref[0] + cnt_ref[1]
    denom = jnp.maximum(cnt, 1.0)
    o_ref[:, :D] = ssum / denom
    o_ref[:, D:] = jnp.where(cnt > 0.0, x_ref[...], 0.0)


def _combine(acc, cnt, x):
    R = 2000
    grid = (N_NODES // R,)
    return pl.pallas_call(
        _combine_body,
        grid=grid,
        in_specs=[
            pl.BlockSpec((NC, R, D), lambda i: (0, i, 0)),
            pl.BlockSpec((NC, R, 1), lambda i: (0, i, 0)),
            pl.BlockSpec((R, D), lambda i: (i, 0)),
        ],
        out_specs=pl.BlockSpec((R, 2 * D), lambda i: (i, 0)),
        out_shape=jax.ShapeDtypeStruct((N_NODES, 2 * D), jnp.float32),
    )(acc, cnt, x)


def kernel(x, es):
    es32 = es.astype(jnp.int32)
    npad = PROWS * B - N_EDGES
    pad_i = jnp.arange(npad, dtype=jnp.int32)
    col_pad = N_NODES + (pad_i % (N_ACC - N_NODES))
    row_pad = pad_i % N_NODES
    col2 = jnp.concatenate([es32[0], col_pad]).reshape(PROWS, B)
    row2 = jnp.concatenate([es32[1], row_pad]).reshape(PROWS, B)
    acc, cnt_pad = _sc_partials(x, col2, row2)
    cnt = cnt_pad[:, :N_NODES].reshape(NC, N_NODES, 1)
    return _combine(acc, cnt, x)
